# Initial kernel scaffold; baseline (speedup 1.0000x reference)
import jax
import jax.numpy as jnp
from jax.experimental import pallas as pl


def kernel(input, edge_index, edge_weight, W, bias):
    raise NotImplementedError("write your pallas kernel here")


# speedup vs baseline: 6.6182x; 6.6182x over previous
import functools

import jax
import jax.numpy as jnp
from jax import lax
from jax.experimental import pallas as pl
from jax.experimental.pallas import tpu as pltpu
from jax.experimental.pallas import tpu_sc as plsc

_L = 16

_GDN = lax.GatherDimensionNumbers(
    offset_dims=(), collapsed_slice_dims=(0,), start_index_map=(0,))


def _bcast_lane(v16, j):
    idx = jnp.full((_L, 1), j, jnp.int32)
    return lax.gather(v16, idx, dimension_numbers=_GDN, slice_sizes=(1,),
                      mode=lax.GatherScatterMode.PROMISE_IN_BOUNDS)


def _matmul(x, w):
    n, d_in = x.shape
    d_out = w.shape[1]
    blk = 1000

    def body(x_ref, w_ref, o_ref):
        o_ref[...] = jnp.dot(x_ref[...], w_ref[...],
                             preferred_element_type=jnp.float32)

    return pl.pallas_call(
        body,
        grid=(n // blk,),
        in_specs=[
            pl.BlockSpec((blk, d_in), lambda i: (i, 0)),
            pl.BlockSpec((d_in, d_out), lambda i: (0, 0)),
        ],
        out_specs=pl.BlockSpec((blk, d_out), lambda i: (i, 0)),
        out_shape=jax.ShapeDtypeStruct((n, d_out), jnp.float32),
    )(x, w)


def _merge(partials, bias):
    _, n, d = partials.shape
    blk = 1000

    def body(p_ref, b_ref, o_ref):
        o_ref[...] = p_ref[0] + p_ref[1] + b_ref[...]

    return pl.pallas_call(
        body,
        grid=(n // blk,),
        in_specs=[
            pl.BlockSpec((2, blk, d), lambda i: (0, i, 0)),
            pl.BlockSpec((1, d), lambda i: (0, 0)),
        ],
        out_specs=pl.BlockSpec((blk, d), lambda i: (i, 0)),
        out_shape=jax.ShapeDtypeStruct((n, d), jnp.float32),
    )(partials, bias)


def _spmm_partials(supp, dst_idx, src_idx, edge_weight):
    n, d = supp.shape
    e = edge_weight.shape[0]
    nw = 32
    ept = e // nw
    k = 80
    nch = ept // k
    rpt = (n // 16) // 8 * 8
    tail = n - 16 * rpt
    zr = rpt // 26
    fpr = d // _L

    mesh = plsc.VectorSubcoreMesh(core_axis_name="c", subcore_axis_name="s")

    @functools.partial(
        pl.kernel,
        out_type=jax.ShapeDtypeStruct((2, n, d), jnp.float32),
        mesh=mesh,
        scratch_types=[
            pltpu.VMEM((ept,), jnp.int32),
            pltpu.VMEM((ept,), jnp.int32),
            pltpu.VMEM((ept,), jnp.float32),
            pltpu.VMEM((k,), jnp.int32),
            pltpu.VMEM((k,), jnp.int32),
            pltpu.VMEM((k, d), jnp.float32),
            pltpu.VMEM((zr, d), jnp.float32),
            pltpu.VMEM_SHARED((n, d), jnp.float32),
            pltpu.SemaphoreType.DMA,
        ],
    )
    def spmm(supp_hbm, dsti_hbm, srci_hbm, ew_hbm, part_hbm,
             src_all, dst_all, w_all, src_chunk, dst_chunk, rows, zrows,
             acc, sem):
        c = lax.axis_index("c")
        s = lax.axis_index("s")
        wid = s * 2 + c
        base = wid * ept

        pltpu.sync_copy(srci_hbm.at[pl.ds(base, ept)], src_all)
        pltpu.sync_copy(dsti_hbm.at[pl.ds(base, ept)], dst_all)
        pltpu.sync_copy(ew_hbm.at[pl.ds(base, ept)], w_all)

        def zfill(i, carry):
            for f in range(fpr):
                zrows[i, pl.ds(f * _L, _L)] = jnp.zeros((_L,), jnp.float32)
            return carry

        lax.fori_loop(0, zr, zfill, 0)
        for r in range(rpt // zr):
            pltpu.sync_copy(zrows, acc.at[pl.ds(s * rpt + r * zr, zr)])

        @pl.when(s == 0)
        def _zero_tail():
            pltpu.sync_copy(zrows.at[pl.ds(0, tail)],
                            acc.at[pl.ds(16 * rpt, tail)])

        plsc.subcore_barrier()

        def chunk(ci, carry):
            cb = ci * k
            for g in range(k // _L):
                src_chunk[pl.ds(g * _L, _L)] = src_all[pl.ds(cb + g * _L, _L)]
                dst_chunk[pl.ds(g * _L, _L)] = dst_all[pl.ds(cb + g * _L, _L)]
            pltpu.async_copy(supp_hbm.at[src_chunk], rows, sem).wait()
            for g in range(k // _L):
                w16 = w_all[pl.ds(cb + g * _L, _L)]
                for j in range(_L):
                    ei = g * _L + j
                    wj = _bcast_lane(w16, j)
                    for f in range(fpr):
                        sl = pl.ds(f * _L, _L)
                        rows[ei, sl] = rows[ei, sl] * wj
            pltpu.sync_copy(rows, acc.at[dst_chunk], add=True)
            return carry

        lax.fori_loop(0, nch, chunk, 0)
        plsc.subcore_barrier()

        sl = pl.ds(s * rpt, rpt)
        pltpu.sync_copy(acc.at[sl], part_hbm.at[c, sl])

        @pl.when(s == 0)
        def _write_tail():
            tl = pl.ds(16 * rpt, tail)
            pltpu.sync_copy(acc.at[tl], part_hbm.at[c, tl])

    return spmm(supp, dst_idx, src_idx, edge_weight)


def kernel(input, edge_index, edge_weight, W, bias):
    supp = _matmul(input, W)
    partials = _spmm_partials(supp, edge_index[0], edge_index[1], edge_weight)
    return _merge(partials, bias)


# speedup vs baseline: 10.8144x; 1.6340x over previous
import functools

import jax
import jax.numpy as jnp
from jax import lax
from jax.experimental import pallas as pl
from jax.experimental.pallas import tpu as pltpu
from jax.experimental.pallas import tpu_sc as plsc

_L = 16

_GDN = lax.GatherDimensionNumbers(
    offset_dims=(), collapsed_slice_dims=(0,), start_index_map=(0,))


def _bcast_lane(v16, j):
    idx = jnp.full((_L, 1), j, jnp.int32)
    return lax.gather(v16, idx, dimension_numbers=_GDN, slice_sizes=(1,),
                      mode=lax.GatherScatterMode.PROMISE_IN_BOUNDS)


def _matmul(x, w):
    n, d_in = x.shape
    d_out = w.shape[1]
    blk = 1000

    def body(x_ref, w_ref, o_ref):
        o_ref[...] = jnp.dot(x_ref[...], w_ref[...],
                             preferred_element_type=jnp.float32)

    return pl.pallas_call(
        body,
        grid=(n // blk,),
        in_specs=[
            pl.BlockSpec((blk, d_in), lambda i: (i, 0)),
            pl.BlockSpec((d_in, d_out), lambda i: (0, 0)),
        ],
        out_specs=pl.BlockSpec((blk, d_out), lambda i: (i, 0)),
        out_shape=jax.ShapeDtypeStruct((n, d_out), jnp.float32),
    )(x, w)


def _merge(partials, bias):
    _, n, d = partials.shape
    blk = 1000

    def body(p_ref, b_ref, o_ref):
        o_ref[...] = p_ref[0] + p_ref[1] + b_ref[...]

    return pl.pallas_call(
        body,
        grid=(n // blk,),
        in_specs=[
            pl.BlockSpec((2, blk, d), lambda i: (0, i, 0)),
            pl.BlockSpec((1, d), lambda i: (0, 0)),
        ],
        out_specs=pl.BlockSpec((blk, d), lambda i: (i, 0)),
        out_shape=jax.ShapeDtypeStruct((n, d), jnp.float32),
    )(partials, bias)


def _spmm_partials(supp, dst_idx, src_idx, edge_weight):
    n, d = supp.shape
    e = edge_weight.shape[0]
    nw = 32
    ept = e // nw
    k = 80
    nch = ept // k
    rpt = (n // 16) // 8 * 8
    tail = n - 16 * rpt
    zr = rpt // 26
    fpr = d // _L

    mesh = plsc.VectorSubcoreMesh(core_axis_name="c", subcore_axis_name="s")

    @functools.partial(
        pl.kernel,
        out_type=jax.ShapeDtypeStruct((2, n, d), jnp.float32),
        mesh=mesh,
        scratch_types=[
            pltpu.VMEM((ept,), jnp.int32),
            pltpu.VMEM((ept,), jnp.float32),
            pltpu.VMEM((k,), jnp.int32),
            pltpu.VMEM((k,), jnp.int32),
            pltpu.VMEM((k, d), jnp.float32),
            pltpu.VMEM((k, d), jnp.float32),
            pltpu.VMEM((zr, d), jnp.float32),
            pltpu.VMEM_SHARED((n, d), jnp.float32),
            pltpu.SemaphoreType.DMA,
            pltpu.SemaphoreType.DMA,
            pltpu.SemaphoreType.DMA,
            pltpu.SemaphoreType.DMA,
        ],
    )
    def spmm(supp_hbm, dsti_hbm, srci_hbm, ew_hbm, part_hbm,
             src_all, w_all, dst_c0, dst_c1, rows0, rows1, zrows,
             acc, g0, g1, s0, s1):
        c = lax.axis_index("c")
        s = lax.axis_index("s")
        wid = s * 2 + c
        base = wid * ept
        rows_ = (rows0, rows1)
        dstc_ = (dst_c0, dst_c1)
        gsem_ = (g0, g1)
        ssem_ = (s0, s1)

        pltpu.sync_copy(srci_hbm.at[pl.ds(base, ept)], src_all)
        pltpu.sync_copy(ew_hbm.at[pl.ds(base, ept)], w_all)

        def zfill(i, carry):
            for f in range(fpr):
                zrows[i, pl.ds(f * _L, _L)] = jnp.zeros((_L,), jnp.float32)
            return carry

        lax.fori_loop(0, zr, zfill, 0)
        for r in range(rpt // zr):
            pltpu.sync_copy(zrows, acc.at[pl.ds(s * rpt + r * zr, zr)])

        @pl.when(s == 0)
        def _zero_tail():
            pltpu.sync_copy(zrows.at[pl.ds(0, tail)],
                            acc.at[pl.ds(16 * rpt, tail)])

        plsc.subcore_barrier()

        def start_gather(ci, b):
            pltpu.async_copy(dsti_hbm.at[pl.ds(base + ci * k, k)],
                             dstc_[b], gsem_[b])
            pltpu.async_copy(supp_hbm.at[src_all.at[pl.ds(ci * k, k)]],
                             rows_[b], gsem_[b])

        def wait_gather(ci, b):
            pltpu.make_async_copy(dsti_hbm.at[pl.ds(base + ci * k, k)],
                                  dstc_[b], gsem_[b]).wait()
            pltpu.make_async_copy(supp_hbm.at[src_all.at[pl.ds(ci * k, k)]],
                                  rows_[b], gsem_[b]).wait()

        def wait_scatter(b):
            pltpu.make_async_copy(rows_[b], acc.at[dstc_[b]], ssem_[b]).wait()

        def process(ci, b):
            wait_gather(ci, b)
            cb = ci * k

            def scale_g(g, carry):
                w16 = w_all[pl.ds(cb + g * _L, _L)]
                for j in range(_L):
                    ei = g * _L + j
                    wj = _bcast_lane(w16, j)
                    for f in range(fpr):
                        sl = pl.ds(f * _L, _L)
                        rows_[b][ei, sl] = rows_[b][ei, sl] * wj
                return carry

            lax.fori_loop(0, k // _L, scale_g, 0)
            pltpu.async_copy(rows_[b], acc.at[dstc_[b]], ssem_[b], add=True)

        start_gather(0, 0)
        start_gather(1, 1)
        process(0, 0)

        def two(i2, carry):
            ci = 1 + 2 * i2
            wait_scatter(0)

            @pl.when(ci + 1 < nch)
            def _():
                start_gather(ci + 1, 0)

            process(ci, 1)
            wait_scatter(1)

            @pl.when(ci + 2 < nch)
            def _():
                start_gather(ci + 2, 1)

            process(ci + 1, 0)
            return carry

        lax.fori_loop(0, (nch - 1) // 2, two, 0)
        wait_scatter(0)
        plsc.subcore_barrier()

        sl = pl.ds(s * rpt, rpt)
        pltpu.sync_copy(acc.at[sl], part_hbm.at[c, sl])

        @pl.when(s == 0)
        def _write_tail():
            tl = pl.ds(16 * rpt, tail)
            pltpu.sync_copy(acc.at[tl], part_hbm.at[c, tl])

    return spmm(supp, dst_idx, src_idx, edge_weight)


def kernel(input, edge_index, edge_weight, W, bias):
    supp = _matmul(input, W)
    partials = _spmm_partials(supp, edge_index[0], edge_index[1], edge_weight)
    return _merge(partials, bias)
